# baseline (speedup 1.0000x reference)
import functools

import jax
import jax.numpy as jnp
from jax import lax
from jax.experimental import pallas as pl
from jax.experimental.pallas import tpu as pltpu
from jax.experimental.pallas import tpu_sc as plsc

R = 8192
S = 128
N = R * S
RES = 128
W = 0.01
RCHUNK = 1024
NSTEP = R // RCHUNK

NC, NS, L = 2, 16, 16
NW = NC * NS
RPW = R // NW
SPW = RPW * S
GPW = RPW // L
STAGE_ROWS = 2048

ICH = 8192
INSTEP = N // ICH


def _tc_body(o_ref, d_ref, aabb_ref, mean_ref,
             tmin_ref, tmax_ref, par_ref, idx_ref, thr_ref, st_ref):
    ox, oy, oz = o_ref[:, 0:1], o_ref[:, 1:2], o_ref[:, 2:3]
    dx, dy, dz = d_ref[:, 0:1], d_ref[:, 1:2], d_ref[:, 2:3]
    nrm = jnp.sqrt(dx * dx + dy * dy + dz * dz)
    dx, dy, dz = dx / nrm, dy / nrm, dz / nrm
    a0x, a0y, a0z = aabb_ref[0, 0], aabb_ref[0, 1], aabb_ref[0, 2]
    a1x, a1y, a1z = aabb_ref[1, 0], aabb_ref[1, 1], aabb_ref[1, 2]
    mean = mean_ref[0, 0]

    def safe(v):
        return jnp.where(jnp.abs(v) < 1e-10, 1e-10, v)

    ix_, iy_, iz_ = 1.0 / safe(dx), 1.0 / safe(dy), 1.0 / safe(dz)
    t1x, t2x = (a0x - ox) * ix_, (a1x - ox) * ix_
    t1y, t2y = (a0y - oy) * iy_, (a1y - oy) * iy_
    t1z, t2z = (a0z - oz) * iz_, (a1z - oz) * iz_
    tmn = jnp.maximum(jnp.maximum(jnp.minimum(t1x, t2x), jnp.minimum(t1y, t2y)),
                      jnp.minimum(t1z, t2z))
    tmx = jnp.minimum(jnp.minimum(jnp.maximum(t1x, t2x), jnp.maximum(t1y, t2y)),
                      jnp.maximum(t1z, t2z))
    hit = tmx > jnp.maximum(tmn, 0.0)
    tmn_c = jnp.minimum(jnp.maximum(tmn, 0.0), 1e10)
    tmx_c = jnp.minimum(tmx, 1e10)
    tmin_ref[...] = tmn_c
    tmax_ref[...] = tmx_c
    span = tmx_c - tmn_c
    step = span / float(S)
    par_ref[:, 0:1] = ox
    par_ref[:, 1:2] = oy
    par_ref[:, 2:3] = oz
    par_ref[:, 3:4] = dx
    par_ref[:, 4:5] = dy
    par_ref[:, 5:6] = dz
    par_ref[:, 6:7] = step
    par_ref[:, 7:8] = jnp.where(hit, 1.0, 0.0)

    j = jax.lax.broadcasted_iota(jnp.int32, (1, S), 1).astype(jnp.float32)
    frac = j / float(S)
    starts = tmn_c + span * frac
    ends = starts + step
    mids = 0.5 * (starts + ends)
    st_ref[...] = starts
    px = ox + dx * mids
    py = oy + dy * mids
    pz = oz + dz * mids
    ux = (px - a0x) / (a1x - a0x)
    uy = (py - a0y) / (a1y - a0y)
    uz = (pz - a0z) / (a1z - a0z)
    gx = jnp.clip((ux * float(RES)).astype(jnp.int32), 0, RES - 1)
    gy = jnp.clip((uy * float(RES)).astype(jnp.int32), 0, RES - 1)
    gz = jnp.clip((uz * float(RES)).astype(jnp.int32), 0, RES - 1)
    idx_ref[...] = (gx * RES + gy) * RES + gz
    delta = jnp.maximum(ends - starts, 1e-10)
    thr = jnp.minimum(W / delta, mean)
    live = hit & (step > 0.0)
    thr_ref[...] = jnp.where(live, thr, jnp.inf)


def _tc_precompute(o, d, aabb, mean11):
    out_shapes = (
        jax.ShapeDtypeStruct((R, 1), jnp.float32),
        jax.ShapeDtypeStruct((R, 1), jnp.float32),
        jax.ShapeDtypeStruct((R, 8), jnp.float32),
        jax.ShapeDtypeStruct((R, S), jnp.int32),
        jax.ShapeDtypeStruct((R, S), jnp.float32),
        jax.ShapeDtypeStruct((R, S), jnp.float32),
    )
    cspec = lambda nc: pl.BlockSpec((RCHUNK, nc), lambda i: (i, 0))
    return pl.pallas_call(
        _tc_body,
        grid=(NSTEP,),
        in_specs=[
            cspec(3), cspec(3),
            pl.BlockSpec(memory_space=pltpu.SMEM),
            pl.BlockSpec(memory_space=pltpu.SMEM),
        ],
        out_specs=(cspec(1), cspec(1), cspec(8), cspec(S), cspec(S), cspec(S)),
        out_shape=out_shapes,
    )(o, d, aabb, mean11)


def _sc_pass1_body(idxF, thrF, grid, validF, counts,
                   idx_v, thr_v, dens_v, cnt_v, sem, gsem):
    wid = lax.axis_index("s") * NC + lax.axis_index("c")
    base_s = wid * SPW
    base_r = wid * RPW
    iota16 = lax.broadcasted_iota(jnp.int32, (L,), 0)

    pltpu.sync_copy(idxF.at[pl.ds(base_s, SPW)], idx_v)
    pltpu.sync_copy(thrF.at[pl.ds(base_s, SPW)], thr_v)

    def gather_chunk(c, carry):
        cps = []
        for t in range(8):
            q = c * 8 + t
            cps.append(pltpu.async_copy(
                grid.at[idx_v.at[pl.ds(q * S, S)]],
                dens_v.at[pl.ds(q * S, S)], gsem))
        for cp in cps:
            cp.wait()
        return carry
    lax.fori_loop(0, RPW // 8, gather_chunk, 0)

    def classify(rb, carry):
        cv = jnp.zeros((L,), jnp.int32)
        for li in range(L):
            rloc = rb * L + li
            acc = jnp.zeros((L,), jnp.int32)
            for q in range(S // L):
                off = rloc * S + q * L
                dv = dens_v[pl.ds(off, L)]
                tv = thr_v[pl.ds(off, L)]
                mi = jnp.where(dv > tv, 1, 0).astype(jnp.int32)
                idx_v[pl.ds(off, L)] = mi
                acc = acc + mi
            cnt = acc[0]
            for i in range(1, L):
                cnt = cnt + acc[i]
            cv = cv + jnp.where(iota16 == li,
                                jnp.full((L,), cnt, jnp.int32), 0)
        cnt_v[pl.ds(rb * L, L)] = cv
        return carry
    lax.fori_loop(0, GPW, classify, 0)

    pltpu.sync_copy(idx_v, validF.at[pl.ds(base_s, SPW)])
    pltpu.sync_copy(cnt_v, counts.at[pl.ds(base_r, RPW)])


def _sc_pass1(idxF, thrF, grid):
    mesh = plsc.VectorSubcoreMesh(core_axis_name="c", subcore_axis_name="s")
    f = functools.partial(
        pl.kernel,
        mesh=mesh,
        out_type=[
            jax.ShapeDtypeStruct((N,), jnp.int32),
            jax.ShapeDtypeStruct((R,), jnp.int32),
        ],
        scratch_types=[
            pltpu.VMEM((SPW,), jnp.int32),
            pltpu.VMEM((SPW,), jnp.float32),
            pltpu.VMEM((SPW,), jnp.float32),
            pltpu.VMEM((RPW,), jnp.int32),
            pltpu.SemaphoreType.DMA,
            pltpu.SemaphoreType.DMA,
        ],
    )(_sc_pass1_body)
    return f(idxF, thrF, grid)


def _sc_pass2_body(validF, counts, paramsF, startsF,
                   c0, c1, c2, c3, c4, c5, c6, c7, info_off, info_cnt,
                   cnt_all, vbuf, sbuf, pbuf, cstage, didx, ioff_v, icnt_v,
                   sem, ssem):
    wid = lax.axis_index("s") * NC + lax.axis_index("c")
    base_s = wid * SPW
    base_r = wid * RPW
    iota16 = lax.broadcasted_iota(jnp.int32, (L,), 0)
    cols = (c0, c1, c2, c3, c4, c5, c6, c7)

    def hsum16(vec):
        acc = vec[0]
        for i in range(1, L):
            acc = acc + vec[i]
        return acc

    pltpu.sync_copy(counts, cnt_all)
    pltpu.sync_copy(validF.at[pl.ds(base_s, SPW)], vbuf)
    pltpu.sync_copy(startsF.at[pl.ds(base_s, SPW)], sbuf)
    pltpu.sync_copy(paramsF.at[pl.ds(base_r * 8, RPW * 8)],
                    pbuf.at[pl.ds(0, RPW * 8)])

    def psum(c, carry):
        tv, bv = carry
        ch = cnt_all[pl.ds(c * L, L)]
        binc = jnp.where(c < wid * GPW, jnp.int32(1), jnp.int32(0))
        return tv + ch, bv + ch * binc
    zero_v = jnp.zeros((L,), jnp.int32)
    tv, bv = lax.fori_loop(0, R // L, psum, (zero_v, zero_v))
    k_total = hsum16(tv)
    base_off = hsum16(bv)
    trash = jnp.minimum(k_total, jnp.int32(N - 1))

    def block(rb, group_base):
        ivo = jnp.zeros((L,), jnp.int32)
        gb = group_base
        cntv = cnt_all[pl.ds(base_r + rb * L, L)]
        for li in range(L):
            rloc = rb * L + li
            prow = pbuf[pl.ds(rloc * 8, L)]
            step_s = prow[6]
            ivo = ivo + jnp.where(iota16 == li,
                                  jnp.full((L,), gb, jnp.int32), 0)
            for q in range(S // L):
                off = rloc * S + q * L
                m = vbuf[pl.ds(off, L)] > 0
                mi = jnp.where(m, 1, 0).astype(jnp.int32)
                st = sbuf[pl.ds(off, L)]
                en = st + step_s
                excl = jnp.zeros((L,), jnp.int32)
                for k in range(L - 1):
                    excl = excl + jnp.where(
                        iota16 > k, jnp.full((L,), mi[k], jnp.int32), 0)
                dest = jnp.where(m, gb + excl, trash)
                didx[0, pl.ds(q * L, L)] = dest
                for c in range(6):
                    cstage[pl.ds(c * S + q * L, L)] = jnp.full(
                        (L,), prow[c], jnp.float32)
                cstage[pl.ds(6 * S + q * L, L)] = st
                cstage[pl.ds(7 * S + q * L, L)] = en
                gb = gb + hsum16(mi)
            cps = []
            for c in range(8):
                cps.append(pltpu.async_copy(
                    cstage.at[pl.ds(c * S, S)],
                    cols[c].at[didx.at[0]], ssem))
            for cp in cps:
                cp.wait()
        ioff_v[pl.ds(rb * L, L)] = ivo
        icnt_v[pl.ds(rb * L, L)] = cntv
        return gb
    lax.fori_loop(0, GPW, block, base_off)

    pltpu.sync_copy(ioff_v, info_off.at[pl.ds(base_r, RPW)])
    pltpu.sync_copy(icnt_v, info_cnt.at[pl.ds(base_r, RPW)])


def _sc_pass2(validF, counts, paramsF, startsF):
    mesh = plsc.VectorSubcoreMesh(core_axis_name="c", subcore_axis_name="s")
    f = functools.partial(
        pl.kernel,
        mesh=mesh,
        out_type=(
            [jax.ShapeDtypeStruct((N,), jnp.float32) for _ in range(8)]
            + [jax.ShapeDtypeStruct((R,), jnp.int32),
               jax.ShapeDtypeStruct((R,), jnp.int32)]),
        scratch_types=[
            pltpu.VMEM((R,), jnp.int32),
            pltpu.VMEM((SPW,), jnp.int32),
            pltpu.VMEM((SPW,), jnp.float32),
            pltpu.VMEM((RPW * 8 + L,), jnp.float32),
            pltpu.VMEM((8 * S,), jnp.float32),
            pltpu.VMEM((1, S), jnp.int32),
            pltpu.VMEM((RPW,), jnp.int32),
            pltpu.VMEM((RPW,), jnp.int32),
            pltpu.SemaphoreType.DMA,
            pltpu.SemaphoreType.DMA,
        ],
    )(_sc_pass2_body)
    return f(validF, counts, paramsF, startsF)


def _tc_inter_body(c0, c1, c2, c3, c4, c5, c6, c7, counts_ref, out_ref):
    pid = pl.program_id(0)
    k_total = jnp.sum(counts_ref[...])
    cols = (c0, c1, c2, c3, c4, c5, c6, c7)
    rows8 = jnp.concatenate([c[...] for c in cols], axis=0)
    rows = jnp.transpose(rows8, (1, 0))
    gidx = jax.lax.broadcasted_iota(jnp.int32, (ICH, 1), 0) + pid * ICH
    out_ref[...] = jnp.where(gidx < k_total, rows, 0.0)


def _tc_interleave(cols, counts):
    return pl.pallas_call(
        _tc_inter_body,
        grid=(INSTEP,),
        in_specs=(
            [pl.BlockSpec((1, ICH), lambda i: (0, i)) for _ in range(8)]
            + [pl.BlockSpec((1, R), lambda i: (0, 0))]),
        out_specs=pl.BlockSpec((ICH, 8), lambda i: (i, 0)),
        out_shape=jax.ShapeDtypeStruct((N, 8), jnp.float32),
    )(*[c.reshape(1, N) for c in cols], counts.reshape(1, R))


def kernel(origins, directions, aabb, density_grid):
    mean11 = jnp.mean(density_grid).reshape(1, 1)
    tmin2, tmax2, params, idxR, thrR, startsR = _tc_precompute(
        origins, directions, aabb, mean11)
    validF, counts = _sc_pass1(idxR.reshape(N), thrR.reshape(N), density_grid)
    outs = _sc_pass2(validF, counts, params.reshape(R * 8), startsR.reshape(N))
    cols, info_off, info_cnt = outs[:8], outs[8], outs[9]
    packed = _tc_interleave(cols, counts)
    packed_info = jnp.stack([info_off, info_cnt], axis=-1)
    return packed, packed_info, tmin2.reshape(R), tmax2.reshape(R)


# speedup vs baseline: 6.3842x; 6.3842x over previous
import functools

import jax
import jax.numpy as jnp
from jax import lax
from jax.experimental import pallas as pl
from jax.experimental.pallas import tpu as pltpu
from jax.experimental.pallas import tpu_sc as plsc

R = 8192
S = 128
N = R * S
RES = 128
W = 0.01
RCHUNK = 1024
NSTEP = R // RCHUNK

NC, NS, L = 2, 16, 16
NW = NC * NS
RPW = R // NW
SPW = RPW * S
GPW = RPW // L
STAGE_ROWS = 2048

ICH = 8192
INSTEP = N // ICH


def _tc_body(o_ref, d_ref, aabb_ref, mean_ref,
             tmin_ref, tmax_ref, par_ref, idx_ref, thr_ref, st_ref):
    ox, oy, oz = o_ref[:, 0:1], o_ref[:, 1:2], o_ref[:, 2:3]
    dx, dy, dz = d_ref[:, 0:1], d_ref[:, 1:2], d_ref[:, 2:3]
    a0x, a0y, a0z = aabb_ref[0, 0], aabb_ref[0, 1], aabb_ref[0, 2]
    a1x, a1y, a1z = aabb_ref[1, 0], aabb_ref[1, 1], aabb_ref[1, 2]
    mean = mean_ref[0, 0]

    def safe(v):
        return jnp.where(jnp.abs(v) < 1e-10, 1e-10, v)

    ix_, iy_, iz_ = 1.0 / safe(dx), 1.0 / safe(dy), 1.0 / safe(dz)
    t1x, t2x = (a0x - ox) * ix_, (a1x - ox) * ix_
    t1y, t2y = (a0y - oy) * iy_, (a1y - oy) * iy_
    t1z, t2z = (a0z - oz) * iz_, (a1z - oz) * iz_
    tmn = jnp.maximum(jnp.maximum(jnp.minimum(t1x, t2x), jnp.minimum(t1y, t2y)),
                      jnp.minimum(t1z, t2z))
    tmx = jnp.minimum(jnp.minimum(jnp.maximum(t1x, t2x), jnp.maximum(t1y, t2y)),
                      jnp.maximum(t1z, t2z))
    hit = tmx > jnp.maximum(tmn, 0.0)
    tmn_c = jnp.minimum(jnp.maximum(tmn, 0.0), 1e10)
    tmx_c = jnp.minimum(tmx, 1e10)
    tmin_ref[...] = tmn_c
    tmax_ref[...] = tmx_c
    span = tmx_c - tmn_c
    step = span / float(S)
    par_ref[:, 0:1] = ox
    par_ref[:, 1:2] = oy
    par_ref[:, 2:3] = oz
    par_ref[:, 3:4] = dx
    par_ref[:, 4:5] = dy
    par_ref[:, 5:6] = dz
    par_ref[:, 6:7] = step
    par_ref[:, 7:8] = jnp.where(hit, 1.0, 0.0)

    j = jax.lax.broadcasted_iota(jnp.int32, (1, S), 1).astype(jnp.float32)
    frac = j / float(S)
    starts = tmn_c + span * frac
    ends = starts + step
    mids = 0.5 * (starts + ends)
    st_ref[...] = starts
    px = ox + dx * mids
    py = oy + dy * mids
    pz = oz + dz * mids
    ux = (px - a0x) / (a1x - a0x)
    uy = (py - a0y) / (a1y - a0y)
    uz = (pz - a0z) / (a1z - a0z)
    gx = jnp.clip((ux * float(RES)).astype(jnp.int32), 0, RES - 1)
    gy = jnp.clip((uy * float(RES)).astype(jnp.int32), 0, RES - 1)
    gz = jnp.clip((uz * float(RES)).astype(jnp.int32), 0, RES - 1)
    idx_ref[...] = (gx * RES + gy) * RES + gz
    delta = jnp.maximum(ends - starts, 1e-10)
    thr = jnp.minimum(W / delta, mean)
    live = hit & (step > 0.0)
    thr_ref[...] = jnp.where(live, thr, jnp.inf)


def _tc_precompute(o, d, aabb, mean11):
    out_shapes = (
        jax.ShapeDtypeStruct((R, 1), jnp.float32),
        jax.ShapeDtypeStruct((R, 1), jnp.float32),
        jax.ShapeDtypeStruct((R, 8), jnp.float32),
        jax.ShapeDtypeStruct((R, S), jnp.int32),
        jax.ShapeDtypeStruct((R, S), jnp.float32),
        jax.ShapeDtypeStruct((R, S), jnp.float32),
    )
    cspec = lambda nc: pl.BlockSpec((RCHUNK, nc), lambda i: (i, 0))
    return pl.pallas_call(
        _tc_body,
        grid=(NSTEP,),
        in_specs=[
            cspec(3), cspec(3),
            pl.BlockSpec(memory_space=pltpu.SMEM),
            pl.BlockSpec(memory_space=pltpu.SMEM),
        ],
        out_specs=(cspec(1), cspec(1), cspec(8), cspec(S), cspec(S), cspec(S)),
        out_shape=out_shapes,
    )(o, d, aabb, mean11)


def _sc_pass1_body(idxF, thrF, grid, validF, counts,
                   idx_v, thr_v, dens_v, cnt_v, sem, gsem):
    wid = lax.axis_index("s") * NC + lax.axis_index("c")
    base_s = wid * SPW
    base_r = wid * RPW
    iota16 = lax.broadcasted_iota(jnp.int32, (L,), 0)

    pltpu.sync_copy(idxF.at[pl.ds(base_s, SPW)], idx_v)
    pltpu.sync_copy(thrF.at[pl.ds(base_s, SPW)], thr_v)

    def gather_chunk(c, carry):
        cps = []
        for t in range(8):
            q = c * 8 + t
            cps.append(pltpu.async_copy(
                grid.at[idx_v.at[pl.ds(q * S, S)]],
                dens_v.at[pl.ds(q * S, S)], gsem))
        for cp in cps:
            cp.wait()
        return carry
    lax.fori_loop(0, RPW // 8, gather_chunk, 0)

    def classify(rb, carry):
        cv = jnp.zeros((L,), jnp.int32)
        for li in range(L):
            rloc = rb * L + li
            acc = jnp.zeros((L,), jnp.int32)
            for q in range(S // L):
                off = rloc * S + q * L
                dv = dens_v[pl.ds(off, L)]
                tv = thr_v[pl.ds(off, L)]
                mi = jnp.where(dv > tv, 1, 0).astype(jnp.int32)
                idx_v[pl.ds(off, L)] = mi
                acc = acc + mi
            cnt = acc[0]
            for i in range(1, L):
                cnt = cnt + acc[i]
            cv = cv + jnp.where(iota16 == li,
                                jnp.full((L,), cnt, jnp.int32), 0)
        cnt_v[pl.ds(rb * L, L)] = cv
        return carry
    lax.fori_loop(0, GPW, classify, 0)

    pltpu.sync_copy(idx_v, validF.at[pl.ds(base_s, SPW)])
    pltpu.sync_copy(cnt_v, counts.at[pl.ds(base_r, RPW)])


def _sc_pass1(idxF, thrF, grid):
    mesh = plsc.VectorSubcoreMesh(core_axis_name="c", subcore_axis_name="s")
    f = functools.partial(
        pl.kernel,
        mesh=mesh,
        out_type=[
            jax.ShapeDtypeStruct((N,), jnp.int32),
            jax.ShapeDtypeStruct((R,), jnp.int32),
        ],
        scratch_types=[
            pltpu.VMEM((SPW,), jnp.int32),
            pltpu.VMEM((SPW,), jnp.float32),
            pltpu.VMEM((SPW,), jnp.float32),
            pltpu.VMEM((RPW,), jnp.int32),
            pltpu.SemaphoreType.DMA,
            pltpu.SemaphoreType.DMA,
        ],
    )(_sc_pass1_body)
    return f(idxF, thrF, grid)


def _sc_pass2_body(validF, counts, paramsF, startsF,
                   c0, c1, c2, c3, c4, c5, c6, c7, info_off, info_cnt,
                   cnt_all, vbuf, sbuf, pbuf, cstage, didx, ioff_v, icnt_v,
                   sem, ssem):
    wid = lax.axis_index("s") * NC + lax.axis_index("c")
    base_s = wid * SPW
    base_r = wid * RPW
    iota16 = lax.broadcasted_iota(jnp.int32, (L,), 0)
    cols = (c0, c1, c2, c3, c4, c5, c6, c7)

    def hsum16(vec):
        acc = vec[0]
        for i in range(1, L):
            acc = acc + vec[i]
        return acc

    pltpu.sync_copy(counts, cnt_all)
    pltpu.sync_copy(validF.at[pl.ds(base_s, SPW)], vbuf)
    pltpu.sync_copy(startsF.at[pl.ds(base_s, SPW)], sbuf)
    pltpu.sync_copy(paramsF.at[pl.ds(base_r * 8, RPW * 8)],
                    pbuf.at[pl.ds(0, RPW * 8)])

    def psum(c, carry):
        tv, bv = carry
        ch = cnt_all[pl.ds(c * L, L)]
        binc = jnp.where(c < wid * GPW, jnp.int32(1), jnp.int32(0))
        return tv + ch, bv + ch * binc
    zero_v = jnp.zeros((L,), jnp.int32)
    tv, bv = lax.fori_loop(0, R // L, psum, (zero_v, zero_v))
    k_total = hsum16(tv)
    base_off = hsum16(bv)

    def block(rb, group_base):
        ivo = jnp.zeros((L,), jnp.int32)
        gb = group_base
        cntv = cnt_all[pl.ds(base_r + rb * L, L)]
        for li in range(L):
            rloc = rb * L + li
            prow = pbuf[pl.ds(rloc * 8, L)]
            step_s = prow[6]
            ivo = ivo + jnp.where(iota16 == li,
                                  jnp.full((L,), gb, jnp.int32), 0)
            for q in range(S // L):
                off = rloc * S + q * L
                m = vbuf[pl.ds(off, L)] > 0
                mi = jnp.where(m, 1, 0).astype(jnp.int32)
                st = sbuf[pl.ds(off, L)]
                en = st + step_s
                excl = jnp.zeros((L,), jnp.int32)
                for k in range(L - 1):
                    excl = excl + jnp.where(
                        iota16 > k, jnp.full((L,), mi[k], jnp.int32), 0)
                p_vec = base_s + rloc * S + q * L + iota16
                vrank = gb + excl
                dest = jnp.where(m, vrank, k_total + p_vec - vrank)
                didx[0, pl.ds(q * L, L)] = dest
                for c in range(6):
                    cstage[pl.ds(c * S + q * L, L)] = jnp.full(
                        (L,), prow[c], jnp.float32)
                cstage[pl.ds(6 * S + q * L, L)] = st
                cstage[pl.ds(7 * S + q * L, L)] = en
                gb = gb + hsum16(mi)
            cps = []
            for c in range(8):
                cps.append(pltpu.async_copy(
                    cstage.at[pl.ds(c * S, S)],
                    cols[c].at[didx.at[0]], ssem))
            for cp in cps:
                cp.wait()
        ioff_v[pl.ds(rb * L, L)] = ivo
        icnt_v[pl.ds(rb * L, L)] = cntv
        return gb
    lax.fori_loop(0, GPW, block, base_off)

    pltpu.sync_copy(ioff_v, info_off.at[pl.ds(base_r, RPW)])
    pltpu.sync_copy(icnt_v, info_cnt.at[pl.ds(base_r, RPW)])


def _sc_pass2(validF, counts, paramsF, startsF):
    mesh = plsc.VectorSubcoreMesh(core_axis_name="c", subcore_axis_name="s")
    f = functools.partial(
        pl.kernel,
        mesh=mesh,
        out_type=(
            [jax.ShapeDtypeStruct((N,), jnp.float32) for _ in range(8)]
            + [jax.ShapeDtypeStruct((R,), jnp.int32),
               jax.ShapeDtypeStruct((R,), jnp.int32)]),
        scratch_types=[
            pltpu.VMEM((R,), jnp.int32),
            pltpu.VMEM((SPW,), jnp.int32),
            pltpu.VMEM((SPW,), jnp.float32),
            pltpu.VMEM((RPW * 8 + L,), jnp.float32),
            pltpu.VMEM((8 * S,), jnp.float32),
            pltpu.VMEM((1, S), jnp.int32),
            pltpu.VMEM((RPW,), jnp.int32),
            pltpu.VMEM((RPW,), jnp.int32),
            pltpu.SemaphoreType.DMA,
            pltpu.SemaphoreType.DMA,
        ],
    )(_sc_pass2_body)
    return f(validF, counts, paramsF, startsF)


def _tc_inter_body(c0, c1, c2, c3, c4, c5, c6, c7, counts_ref, out_ref):
    pid = pl.program_id(0)
    k_total = jnp.sum(counts_ref[...])
    cols = (c0, c1, c2, c3, c4, c5, c6, c7)
    rows8 = jnp.concatenate([c[...] for c in cols], axis=0)
    rows = jnp.transpose(rows8, (1, 0))
    gidx = jax.lax.broadcasted_iota(jnp.int32, (ICH, 1), 0) + pid * ICH
    out_ref[...] = jnp.where(gidx < k_total, rows, 0.0)


def _tc_interleave(cols, counts):
    return pl.pallas_call(
        _tc_inter_body,
        grid=(INSTEP,),
        in_specs=(
            [pl.BlockSpec((1, ICH), lambda i: (0, i)) for _ in range(8)]
            + [pl.BlockSpec((1, R), lambda i: (0, 0))]),
        out_specs=pl.BlockSpec((ICH, 8), lambda i: (i, 0)),
        out_shape=jax.ShapeDtypeStruct((N, 8), jnp.float32),
    )(*[c.reshape(1, N) for c in cols], counts.reshape(1, R))


def kernel(origins, directions, aabb, density_grid):
    mean11 = jnp.mean(density_grid).reshape(1, 1)
    dn = directions / jnp.linalg.norm(directions, axis=-1, keepdims=True)
    tmin2, tmax2, params, idxR, thrR, startsR = _tc_precompute(
        origins, dn, aabb, mean11)
    validF, counts = _sc_pass1(idxR.reshape(N), thrR.reshape(N), density_grid)
    outs = _sc_pass2(validF, counts, params.reshape(R * 8), startsR.reshape(N))
    cols, info_off, info_cnt = outs[:8], outs[8], outs[9]
    packed = _tc_interleave(cols, counts)
    packed_info = jnp.stack([info_off, info_cnt], axis=-1)
    return packed, packed_info, tmin2.reshape(R), tmax2.reshape(R)


# speedup vs baseline: 6.3948x; 1.0017x over previous
import functools

import jax
import jax.numpy as jnp
from jax import lax
from jax.experimental import pallas as pl
from jax.experimental.pallas import tpu as pltpu
from jax.experimental.pallas import tpu_sc as plsc

R = 8192
S = 128
N = R * S
RES = 128
W = 0.01
RCHUNK = 1024
NSTEP = R // RCHUNK

NC, NS, L = 2, 16, 16
NW = NC * NS
RPW = R // NW
SPW = RPW * S
GPW = RPW // L
STAGE_ROWS = 2048

ICH = 8192
INSTEP = N // ICH


def _tc_body(o_ref, d_ref, aabb_ref, mean_ref,
             tmin_ref, tmax_ref, par_ref, idx_ref, thr_ref, st_ref):
    ox, oy, oz = o_ref[:, 0:1], o_ref[:, 1:2], o_ref[:, 2:3]
    dx, dy, dz = d_ref[:, 0:1], d_ref[:, 1:2], d_ref[:, 2:3]
    a0x, a0y, a0z = aabb_ref[0, 0], aabb_ref[0, 1], aabb_ref[0, 2]
    a1x, a1y, a1z = aabb_ref[1, 0], aabb_ref[1, 1], aabb_ref[1, 2]
    mean = mean_ref[0, 0]

    def safe(v):
        return jnp.where(jnp.abs(v) < 1e-10, 1e-10, v)

    ix_, iy_, iz_ = 1.0 / safe(dx), 1.0 / safe(dy), 1.0 / safe(dz)
    t1x, t2x = (a0x - ox) * ix_, (a1x - ox) * ix_
    t1y, t2y = (a0y - oy) * iy_, (a1y - oy) * iy_
    t1z, t2z = (a0z - oz) * iz_, (a1z - oz) * iz_
    tmn = jnp.maximum(jnp.maximum(jnp.minimum(t1x, t2x), jnp.minimum(t1y, t2y)),
                      jnp.minimum(t1z, t2z))
    tmx = jnp.minimum(jnp.minimum(jnp.maximum(t1x, t2x), jnp.maximum(t1y, t2y)),
                      jnp.maximum(t1z, t2z))
    hit = tmx > jnp.maximum(tmn, 0.0)
    tmn_c = jnp.minimum(jnp.maximum(tmn, 0.0), 1e10)
    tmx_c = jnp.minimum(tmx, 1e10)
    tmin_ref[...] = tmn_c
    tmax_ref[...] = tmx_c
    span = tmx_c - tmn_c
    step = span / float(S)
    par_ref[:, 0:1] = ox
    par_ref[:, 1:2] = oy
    par_ref[:, 2:3] = oz
    par_ref[:, 3:4] = dx
    par_ref[:, 4:5] = dy
    par_ref[:, 5:6] = dz
    par_ref[:, 6:7] = step
    par_ref[:, 7:8] = jnp.where(hit, 1.0, 0.0)

    j = jax.lax.broadcasted_iota(jnp.int32, (1, S), 1).astype(jnp.float32)
    frac = j / float(S)
    starts = tmn_c + span * frac
    ends = starts + step
    mids = 0.5 * (starts + ends)
    st_ref[...] = starts
    px = ox + dx * mids
    py = oy + dy * mids
    pz = oz + dz * mids
    ux = (px - a0x) / (a1x - a0x)
    uy = (py - a0y) / (a1y - a0y)
    uz = (pz - a0z) / (a1z - a0z)
    gx = jnp.clip((ux * float(RES)).astype(jnp.int32), 0, RES - 1)
    gy = jnp.clip((uy * float(RES)).astype(jnp.int32), 0, RES - 1)
    gz = jnp.clip((uz * float(RES)).astype(jnp.int32), 0, RES - 1)
    idx_ref[...] = (gx * RES + gy) * RES + gz
    delta = jnp.maximum(ends - starts, 1e-10)
    thr = jnp.minimum(W / delta, mean)
    live = hit & (step > 0.0)
    thr_ref[...] = jnp.where(live, thr, jnp.inf)


def _tc_precompute(o, d, aabb, mean11):
    out_shapes = (
        jax.ShapeDtypeStruct((R, 1), jnp.float32),
        jax.ShapeDtypeStruct((R, 1), jnp.float32),
        jax.ShapeDtypeStruct((R, 8), jnp.float32),
        jax.ShapeDtypeStruct((R, S), jnp.int32),
        jax.ShapeDtypeStruct((R, S), jnp.float32),
        jax.ShapeDtypeStruct((R, S), jnp.float32),
    )
    cspec = lambda nc: pl.BlockSpec((RCHUNK, nc), lambda i: (i, 0))
    return pl.pallas_call(
        _tc_body,
        grid=(NSTEP,),
        in_specs=[
            cspec(3), cspec(3),
            pl.BlockSpec(memory_space=pltpu.SMEM),
            pl.BlockSpec(memory_space=pltpu.SMEM),
        ],
        out_specs=(cspec(1), cspec(1), cspec(8), cspec(S), cspec(S), cspec(S)),
        out_shape=out_shapes,
    )(o, d, aabb, mean11)


def _sc_pass1_body(idxF, thrF, grid, validF, counts,
                   idx_v, thr_v, dens_v, cnt_v, sem, gsem):
    wid = lax.axis_index("s") * NC + lax.axis_index("c")
    base_s = wid * SPW
    base_r = wid * RPW
    iota16 = lax.broadcasted_iota(jnp.int32, (L,), 0)

    pltpu.sync_copy(idxF.at[pl.ds(base_s, SPW)], idx_v)
    pltpu.sync_copy(thrF.at[pl.ds(base_s, SPW)], thr_v)

    def gather_chunk(c, carry):
        cps = []
        for t in range(8):
            q = c * 8 + t
            cps.append(pltpu.async_copy(
                grid.at[idx_v.at[pl.ds(q * S, S)]],
                dens_v.at[pl.ds(q * S, S)], gsem))
        for cp in cps:
            cp.wait()
        return carry
    lax.fori_loop(0, RPW // 8, gather_chunk, 0)

    def classify(rb, carry):
        cv = jnp.zeros((L,), jnp.int32)
        for li in range(L):
            rloc = rb * L + li
            acc = jnp.zeros((L,), jnp.int32)
            for q in range(S // L):
                off = rloc * S + q * L
                dv = dens_v[pl.ds(off, L)]
                tv = thr_v[pl.ds(off, L)]
                mi = jnp.where(dv > tv, 1, 0).astype(jnp.int32)
                idx_v[pl.ds(off, L)] = mi
                acc = acc + mi
            cnt = acc[0]
            for i in range(1, L):
                cnt = cnt + acc[i]
            cv = cv + jnp.where(iota16 == li,
                                jnp.full((L,), cnt, jnp.int32), 0)
        cnt_v[pl.ds(rb * L, L)] = cv
        return carry
    lax.fori_loop(0, GPW, classify, 0)

    pltpu.sync_copy(idx_v, validF.at[pl.ds(base_s, SPW)])
    pltpu.sync_copy(cnt_v, counts.at[pl.ds(base_r, RPW)])


def _sc_pass1(idxF, thrF, grid):
    mesh = plsc.VectorSubcoreMesh(core_axis_name="c", subcore_axis_name="s")
    f = functools.partial(
        pl.kernel,
        mesh=mesh,
        out_type=[
            jax.ShapeDtypeStruct((N,), jnp.int32),
            jax.ShapeDtypeStruct((R,), jnp.int32),
        ],
        scratch_types=[
            pltpu.VMEM((SPW,), jnp.int32),
            pltpu.VMEM((SPW,), jnp.float32),
            pltpu.VMEM((SPW,), jnp.float32),
            pltpu.VMEM((RPW,), jnp.int32),
            pltpu.SemaphoreType.DMA,
            pltpu.SemaphoreType.DMA,
        ],
    )(_sc_pass1_body)
    return f(idxF, thrF, grid)


def _sc_pass2_body(validF, counts, paramsF, startsF,
                   c0, c1, c2, c3, c4, c5, c6, c7, info_off, info_cnt,
                   cnt_all, vbuf, sbuf, pbuf, cstage, didx, ioff_v, icnt_v,
                   sem, ssem):
    wid = lax.axis_index("s") * NC + lax.axis_index("c")
    base_s = wid * SPW
    base_r = wid * RPW
    iota16 = lax.broadcasted_iota(jnp.int32, (L,), 0)
    cols = (c0, c1, c2, c3, c4, c5, c6, c7)

    def hsum16(vec):
        acc = vec[0]
        for i in range(1, L):
            acc = acc + vec[i]
        return acc

    pltpu.sync_copy(counts, cnt_all)
    pltpu.sync_copy(validF.at[pl.ds(base_s, SPW)], vbuf)
    pltpu.sync_copy(startsF.at[pl.ds(base_s, SPW)], sbuf)
    pltpu.sync_copy(paramsF.at[pl.ds(base_r * 8, RPW * 8)],
                    pbuf.at[pl.ds(0, RPW * 8)])

    def psum(c, carry):
        tv, bv = carry
        ch = cnt_all[pl.ds(c * L, L)]
        binc = jnp.where(c < wid * GPW, jnp.int32(1), jnp.int32(0))
        return tv + ch, bv + ch * binc
    zero_v = jnp.zeros((L,), jnp.int32)
    tv, bv = lax.fori_loop(0, R // L, psum, (zero_v, zero_v))
    k_total = hsum16(tv)
    base_off = hsum16(bv)

    def block(rb, group_base):
        pend = [[], []]
        ivo = jnp.zeros((L,), jnp.int32)
        gb = group_base
        cntv = cnt_all[pl.ds(base_r + rb * L, L)]
        for li in range(L):
            rloc = rb * L + li
            prow = pbuf[pl.ds(rloc * 8, L)]
            step_s = prow[6]
            ivo = ivo + jnp.where(iota16 == li,
                                  jnp.full((L,), gb, jnp.int32), 0)
            for q in range(S // L):
                off = rloc * S + q * L
                m = vbuf[pl.ds(off, L)] > 0
                mi = jnp.where(m, 1, 0).astype(jnp.int32)
                st = sbuf[pl.ds(off, L)]
                en = st + step_s
                excl = jnp.zeros((L,), jnp.int32)
                for k in range(L - 1):
                    excl = excl + jnp.where(
                        iota16 > k, jnp.full((L,), mi[k], jnp.int32), 0)
                p_vec = base_s + rloc * S + q * L + iota16
                vrank = gb + excl
                dest = jnp.where(m, vrank, k_total + p_vec - vrank)
                didx[li % 2, pl.ds(q * L, L)] = dest
                for c in range(6):
                    cstage[pl.ds((li % 2) * 8 * S + c * S + q * L, L)] = \
                        jnp.full((L,), prow[c], jnp.float32)
                sb = (li % 2) * 8 * S
                cstage[pl.ds(sb + 6 * S + q * L, L)] = st
                cstage[pl.ds(sb + 7 * S + q * L, L)] = en
                gb = gb + hsum16(mi)
            for cp in pend[li % 2]:
                cp.wait()
            cps = []
            for c in range(8):
                cps.append(pltpu.async_copy(
                    cstage.at[pl.ds((li % 2) * 8 * S + c * S, S)],
                    cols[c].at[didx.at[li % 2]], ssem))
            pend[li % 2] = cps
        for cps in pend:
            for cp in cps:
                cp.wait()
        ioff_v[pl.ds(rb * L, L)] = ivo
        icnt_v[pl.ds(rb * L, L)] = cntv
        return gb
    lax.fori_loop(0, GPW, block, base_off)

    pltpu.sync_copy(ioff_v, info_off.at[pl.ds(base_r, RPW)])
    pltpu.sync_copy(icnt_v, info_cnt.at[pl.ds(base_r, RPW)])


def _sc_pass2(validF, counts, paramsF, startsF):
    mesh = plsc.VectorSubcoreMesh(core_axis_name="c", subcore_axis_name="s")
    f = functools.partial(
        pl.kernel,
        mesh=mesh,
        out_type=(
            [jax.ShapeDtypeStruct((N,), jnp.float32) for _ in range(8)]
            + [jax.ShapeDtypeStruct((R,), jnp.int32),
               jax.ShapeDtypeStruct((R,), jnp.int32)]),
        scratch_types=[
            pltpu.VMEM((R,), jnp.int32),
            pltpu.VMEM((SPW,), jnp.int32),
            pltpu.VMEM((SPW,), jnp.float32),
            pltpu.VMEM((RPW * 8 + L,), jnp.float32),
            pltpu.VMEM((16 * S,), jnp.float32),
            pltpu.VMEM((2, S), jnp.int32),
            pltpu.VMEM((RPW,), jnp.int32),
            pltpu.VMEM((RPW,), jnp.int32),
            pltpu.SemaphoreType.DMA,
            pltpu.SemaphoreType.DMA,
        ],
    )(_sc_pass2_body)
    return f(validF, counts, paramsF, startsF)


def _tc_inter_body(c0, c1, c2, c3, c4, c5, c6, c7, counts_ref, out_ref):
    pid = pl.program_id(0)
    k_total = jnp.sum(counts_ref[...])
    cols = (c0, c1, c2, c3, c4, c5, c6, c7)
    rows8 = jnp.concatenate([c[...] for c in cols], axis=0)
    rows = jnp.transpose(rows8, (1, 0))
    gidx = jax.lax.broadcasted_iota(jnp.int32, (ICH, 1), 0) + pid * ICH
    out_ref[...] = jnp.where(gidx < k_total, rows, 0.0)


def _tc_interleave(cols, counts):
    return pl.pallas_call(
        _tc_inter_body,
        grid=(INSTEP,),
        in_specs=(
            [pl.BlockSpec((1, ICH), lambda i: (0, i)) for _ in range(8)]
            + [pl.BlockSpec((1, R), lambda i: (0, 0))]),
        out_specs=pl.BlockSpec((ICH, 8), lambda i: (i, 0)),
        out_shape=jax.ShapeDtypeStruct((N, 8), jnp.float32),
    )(*[c.reshape(1, N) for c in cols], counts.reshape(1, R))


def kernel(origins, directions, aabb, density_grid):
    mean11 = jnp.mean(density_grid).reshape(1, 1)
    dn = directions / jnp.linalg.norm(directions, axis=-1, keepdims=True)
    tmin2, tmax2, params, idxR, thrR, startsR = _tc_precompute(
        origins, dn, aabb, mean11)
    validF, counts = _sc_pass1(idxR.reshape(N), thrR.reshape(N), density_grid)
    outs = _sc_pass2(validF, counts, params.reshape(R * 8), startsR.reshape(N))
    cols, info_off, info_cnt = outs[:8], outs[8], outs[9]
    packed = _tc_interleave(cols, counts)
    packed_info = jnp.stack([info_off, info_cnt], axis=-1)
    return packed, packed_info, tmin2.reshape(R), tmax2.reshape(R)
